# Initial kernel scaffold; baseline (speedup 1.0000x reference)
import jax
import jax.numpy as jnp
from jax.experimental import pallas as pl


def kernel(batched_pts, W, b, gamma, beta, bn_mean, bn_var):
    raise NotImplementedError("write your pallas kernel here")


# speedup vs baseline: 30.5722x; 30.5722x over previous
import functools

import jax
import jax.numpy as jnp
from jax import lax
from jax.experimental import pallas as pl
from jax.experimental.pallas import tpu as pltpu
from jax.experimental.pallas import tpu_sc as plsc

VX, VY = 1.0, 1.0
XMIN, YMIN, ZMIN = 0.0, -39.68, -3.0
XMAX, YMAX, ZMAX = 69.12, 39.68, 1.0
NX, NY = 69, 79
NP = NX * NY
MAXPTS = 16
COUT = 64

BB = 4
NPTS = 120000
NPAD = 120320
CH = 3760
NCH = NPAD // CH
VECS = CH // 16

RNG = 768
NRNG = 8
PPAD = RNG * NRNG


def _pid_body(pts_ref, pid_ref):
    x = pts_ref[0, 0]
    y = pts_ref[0, 1]
    z = pts_ref[0, 2]
    ixf = jnp.floor((x - XMIN) / VX)
    iyf = jnp.floor((y - YMIN) / VY)
    ix = ixf.astype(jnp.int32)
    iy = iyf.astype(jnp.int32)
    valid = ((ix >= 0) & (ix < NX) & (iy >= 0) & (iy < NY)
             & (z >= ZMIN) & (z < ZMAX))
    pid_ref[0] = jnp.where(valid, iy * NX + ix, NP)


def _compute_pids(pts4d):
    return pl.pallas_call(
        _pid_body,
        grid=(BB,),
        in_specs=[pl.BlockSpec((1, 4, NPAD // 128, 128),
                               lambda b: (b, 0, 0, 0))],
        out_specs=pl.BlockSpec((1, NPAD // 128, 128), lambda b: (b, 0, 0)),
        out_shape=jax.ShapeDtypeStruct((BB, NPAD // 128, 128), jnp.int32),
    )(pts4d)


def _sc_body(pid_hbm, x_hbm, y_hbm, z_hbm, buf_hbm, counts_hbm,
             pidA, pidB, xA, xB, yA, yB, zA, zB,
             counter, valx, valy, valz, semA, semB):
    c = lax.axis_index("c")
    s = lax.axis_index("s")
    b = c * 2 + s // 8
    j = s % 8
    base_p = j * RNG
    pt_base = b * NPAD
    cnt_base = b * PPAD + base_p

    zero16 = jnp.zeros((16,), jnp.int32)
    iota16 = lax.iota(jnp.int32, 16)

    def _zero_counter(k, _):
        counter[pl.ds(k * 16, 16)] = zero16
        return 0
    lax.fori_loop(0, RNG // 16, _zero_counter, 0)

    cal, _ = plsc.scan_count(zero16)
    cbase = jnp.min(cal)

    def _start(chunk, pid_st, x_st, y_st, z_st, sem):
        off = pt_base + chunk * CH
        pltpu.async_copy(pid_hbm.at[pl.ds(off, CH)], pid_st, sem)
        pltpu.async_copy(x_hbm.at[pl.ds(off, CH)], x_st, sem)
        pltpu.async_copy(y_hbm.at[pl.ds(off, CH)], y_st, sem)
        pltpu.async_copy(z_hbm.at[pl.ds(off, CH)], z_st, sem)

    def _drain(pid_st, x_st, y_st, z_st, sem):
        pltpu.make_async_copy(pid_hbm.at[pl.ds(0, CH)], pid_st, sem).wait()
        pltpu.make_async_copy(x_hbm.at[pl.ds(0, CH)], x_st, sem).wait()
        pltpu.make_async_copy(y_hbm.at[pl.ds(0, CH)], y_st, sem).wait()
        pltpu.make_async_copy(z_hbm.at[pl.ds(0, CH)], z_st, sem).wait()

    def _process(pid_st, x_st, y_st, z_st):
        def _vec(v, _):
            pidv = pid_st[pl.ds(v * 16, 16)]
            ploc = pidv - base_p
            inr = (ploc >= 0) & (ploc < RNG)
            psafe = jnp.where(inr, ploc, 0)
            cnt, last = plsc.scan_count(pidv, inr)
            c0 = plsc.load_gather(counter, [psafe], mask=inr)
            rank = c0 + cnt - cbase
            keep = inr & (rank < MAXPTS)
            rsafe = jnp.where(keep, rank, 0)
            xv = x_st[pl.ds(v * 16, 16)]
            yv = y_st[pl.ds(v * 16, 16)]
            zv = z_st[pl.ds(v * 16, 16)]
            plsc.store_scatter(valx, [rsafe, psafe], xv, mask=keep)
            plsc.store_scatter(valy, [rsafe, psafe], yv, mask=keep)
            plsc.store_scatter(valz, [rsafe, psafe], zv, mask=keep)
            plsc.store_scatter(counter, [psafe], c0 + cnt + 1 - cbase,
                               mask=last & inr)
            return 0
        lax.fori_loop(0, VECS, _vec, 0)

    _start(0, pidA, xA, yA, zA, semA)

    def _pair(i, _):
        _drain(pidA, xA, yA, zA, semA)
        _start(2 * i + 1, pidB, xB, yB, zB, semB)
        _process(pidA, xA, yA, zA)
        _drain(pidB, xB, yB, zB, semB)

        @pl.when(i < NCH // 2 - 1)
        def _():
            _start(2 * i + 2, pidA, xA, yA, zA, semA)
        _process(pidB, xB, yB, zB)
        return 0
    lax.fori_loop(0, NCH // 2, _pair, 0)

    def _cap(k, _):
        v = counter[pl.ds(k * 16, 16)]
        counter[pl.ds(k * 16, 16)] = jnp.minimum(v, MAXPTS)
        return 0
    lax.fori_loop(0, RNG // 16, _cap, 0)
    pltpu.sync_copy(counter, counts_hbm.at[pl.ds(cnt_base, RNG)])

    pltpu.sync_copy(valx, buf_hbm.at[b, 0, :, pl.ds(base_p, RNG)])
    pltpu.sync_copy(valy, buf_hbm.at[b, 1, :, pl.ds(base_p, RNG)])
    pltpu.sync_copy(valz, buf_hbm.at[b, 2, :, pl.ds(base_p, RNG)])


def _sc_scatter(pid1, xf, yf, zf):
    mesh = plsc.VectorSubcoreMesh(core_axis_name="c", subcore_axis_name="s")
    f = functools.partial(
        pl.kernel, mesh=mesh,
        out_type=(
            jax.ShapeDtypeStruct((BB, 3, MAXPTS, PPAD), jnp.float32),
            jax.ShapeDtypeStruct((BB * PPAD,), jnp.int32),
        ),
        scratch_types=[
            pltpu.VMEM((CH,), jnp.int32),
            pltpu.VMEM((CH,), jnp.int32),
            pltpu.VMEM((CH,), jnp.float32),
            pltpu.VMEM((CH,), jnp.float32),
            pltpu.VMEM((CH,), jnp.float32),
            pltpu.VMEM((CH,), jnp.float32),
            pltpu.VMEM((CH,), jnp.float32),
            pltpu.VMEM((CH,), jnp.float32),
            pltpu.VMEM((RNG,), jnp.int32),
            pltpu.VMEM((MAXPTS, RNG), jnp.float32),
            pltpu.VMEM((MAXPTS, RNG), jnp.float32),
            pltpu.VMEM((MAXPTS, RNG), jnp.float32),
            pltpu.SemaphoreType.DMA,
            pltpu.SemaphoreType.DMA,
        ],
        compiler_params=pltpu.CompilerParams(needs_layout_passes=False),
    )(_sc_body)
    return f(pid1, xf, yf, zf)


def _enc_body(buf_ref, cnt_ref, xc_ref, yc_ref, w_ref, b_ref, gam_ref,
              bet_ref, mu_ref, var_ref, out_ref):
    x = buf_ref[0, 0]
    y = buf_ref[0, 1]
    z = buf_ref[0, 2]
    cnt2 = cnt_ref[0, 0]
    slot = jax.lax.broadcasted_iota(jnp.int32, (MAXPTS, RNG), 0)
    mask = slot < cnt2
    cntf = jnp.maximum(cnt2, 1).astype(jnp.float32)
    mx = jnp.sum(jnp.where(mask, x, 0.0), 0, keepdims=True) / cntf
    my = jnp.sum(jnp.where(mask, y, 0.0), 0, keepdims=True) / cntf
    mz = jnp.sum(jnp.where(mask, z, 0.0), 0, keepdims=True) / cntf
    xc = xc_ref[0]
    yc = yc_ref[0]
    nonzero = cnt2 > 0
    neg = jnp.float32(-1e9)
    for ci in range(COUT):
        sc_ = gam_ref[0, ci] / jnp.sqrt(var_ref[0, ci] + 1e-5)
        w3 = w_ref[3, ci] * sc_
        w4 = w_ref[4, ci] * sc_
        w5 = w_ref[5, ci] * sc_
        w6 = w_ref[6, ci] * sc_
        w7 = w_ref[7, ci] * sc_
        a0 = w_ref[0, ci] * sc_ + w3 + w6
        a1 = w_ref[1, ci] * sc_ + w4 + w7
        a2 = w_ref[2, ci] * sc_ + w5
        cc = (b_ref[0, ci] * sc_ + bet_ref[0, ci] - mu_ref[0, ci] * sc_
              - w3 * mx - w4 * my - w5 * mz - w6 * xc - w7 * yc)
        h = a0 * x + a1 * y + a2 * z
        m = jnp.max(jnp.where(mask, h, neg), 0, keepdims=True)
        res = jnp.where(nonzero, jnp.maximum(m + cc, 0.0), 0.0)
        out_ref[0, ci] = res[0]


def _encode(buf, counts4, xc3, yc3, w, bb, gamma, beta, mu, var):
    smem = pl.BlockSpec(memory_space=pltpu.SMEM)
    return pl.pallas_call(
        _enc_body,
        grid=(BB, NRNG),
        in_specs=[
            pl.BlockSpec((1, 3, MAXPTS, RNG), lambda b, t: (b, 0, 0, t)),
            pl.BlockSpec((1, 1, 1, RNG), lambda b, t: (b, t, 0, 0)),
            pl.BlockSpec((1, 1, RNG), lambda b, t: (t, 0, 0)),
            pl.BlockSpec((1, 1, RNG), lambda b, t: (t, 0, 0)),
            smem, smem, smem, smem, smem, smem,
        ],
        out_specs=pl.BlockSpec((1, COUT, RNG), lambda b, t: (b, 0, t)),
        out_shape=jax.ShapeDtypeStruct((BB, COUT, PPAD), jnp.float32),
    )(buf, counts4, xc3, yc3, w, bb, gamma, beta, mu, var)


def kernel(batched_pts, W, b, gamma, beta, bn_mean, bn_var):
    ptsT = jnp.pad(batched_pts.transpose(0, 2, 1),
                   ((0, 0), (0, 0), (0, NPAD - NPTS)),
                   constant_values=-1e9)
    pts4d = ptsT.reshape(BB, 4, NPAD // 128, 128)

    pid1 = _compute_pids(pts4d).reshape(BB * NPAD)
    xf = ptsT[:, 0, :].reshape(BB * NPAD)
    yf = ptsT[:, 1, :].reshape(BB * NPAD)
    zf = ptsT[:, 2, :].reshape(BB * NPAD)

    buf, counts = _sc_scatter(pid1, xf, yf, zf)

    p_idx = jnp.arange(PPAD, dtype=jnp.int32)
    xc = ((p_idx % NX).astype(jnp.float32) + 0.5) * VX + XMIN
    yc = ((p_idx // NX).astype(jnp.float32) + 0.5) * VY + YMIN

    out = _encode(buf, counts.reshape(BB, NRNG, 1, RNG),
                  xc.reshape(NRNG, 1, RNG), yc.reshape(NRNG, 1, RNG),
                  W, b.reshape(1, COUT), gamma.reshape(1, COUT),
                  beta.reshape(1, COUT), bn_mean.reshape(1, COUT),
                  bn_var.reshape(1, COUT))

    return out[:, :, :NP].reshape(BB, COUT, NY, NX)


# speedup vs baseline: 31.7577x; 1.0388x over previous
import functools

import jax
import jax.numpy as jnp
from jax import lax
from jax.experimental import pallas as pl
from jax.experimental.pallas import tpu as pltpu
from jax.experimental.pallas import tpu_sc as plsc

VX, VY = 1.0, 1.0
XMIN, YMIN, ZMIN = 0.0, -39.68, -3.0
XMAX, YMAX, ZMAX = 69.12, 39.68, 1.0
NX, NY = 69, 79
NP = NX * NY
MAXPTS = 16
COUT = 64

BB = 4
NPTS = 120000
NPAD = 120320
CH = 3760
NCH = NPAD // CH
VECS = CH // 16

RNG = 768
NRNG = 8
PPAD = RNG * NRNG


def _pid_body(pts_ref, pid_ref):
    x = pts_ref[0, 0]
    y = pts_ref[0, 1]
    z = pts_ref[0, 2]
    ixf = jnp.floor((x - XMIN) / VX)
    iyf = jnp.floor((y - YMIN) / VY)
    ix = ixf.astype(jnp.int32)
    iy = iyf.astype(jnp.int32)
    valid = ((ix >= 0) & (ix < NX) & (iy >= 0) & (iy < NY)
             & (z >= ZMIN) & (z < ZMAX))
    pid_ref[0] = jnp.where(valid, iy * NX + ix, NP)


def _compute_pids(pts4d):
    return pl.pallas_call(
        _pid_body,
        grid=(BB,),
        in_specs=[pl.BlockSpec((1, 4, NPAD // 128, 128),
                               lambda b: (b, 0, 0, 0))],
        out_specs=pl.BlockSpec((1, NPAD // 128, 128), lambda b: (b, 0, 0)),
        out_shape=jax.ShapeDtypeStruct((BB, NPAD // 128, 128), jnp.int32),
    )(pts4d)


def _sc_body(pid_hbm, x_hbm, y_hbm, z_hbm, buf_hbm, counts_hbm,
             pidA, pidB, xA, xB, yA, yB, zA, zB,
             counter, valx, valy, valz, semA, semB):
    c = lax.axis_index("c")
    s = lax.axis_index("s")
    b = c * 2 + s // 8
    j = s % 8
    base_p = j * RNG
    pt_base = b * NPAD
    cnt_base = b * PPAD + base_p

    zero16 = jnp.zeros((16,), jnp.int32)
    iota16 = lax.iota(jnp.int32, 16)

    def _zero_counter(k, _):
        counter[pl.ds(k * 16, 16)] = zero16
        return 0
    lax.fori_loop(0, RNG // 16, _zero_counter, 0)

    cal, _ = plsc.scan_count(zero16)
    cbase = jnp.min(cal)

    def _start(chunk, pid_st, x_st, y_st, z_st, sem):
        off = pt_base + chunk * CH
        pltpu.async_copy(pid_hbm.at[pl.ds(off, CH)], pid_st, sem)
        pltpu.async_copy(x_hbm.at[pl.ds(off, CH)], x_st, sem)
        pltpu.async_copy(y_hbm.at[pl.ds(off, CH)], y_st, sem)
        pltpu.async_copy(z_hbm.at[pl.ds(off, CH)], z_st, sem)

    def _drain(pid_st, x_st, y_st, z_st, sem):
        pltpu.make_async_copy(pid_hbm.at[pl.ds(0, CH)], pid_st, sem).wait()
        pltpu.make_async_copy(x_hbm.at[pl.ds(0, CH)], x_st, sem).wait()
        pltpu.make_async_copy(y_hbm.at[pl.ds(0, CH)], y_st, sem).wait()
        pltpu.make_async_copy(z_hbm.at[pl.ds(0, CH)], z_st, sem).wait()

    def _process(pid_st, x_st, y_st, z_st):
        def _vec(v, _):
            pidv = pid_st[pl.ds(v * 16, 16)]
            ploc = pidv - base_p
            inr = (ploc >= 0) & (ploc < RNG)
            psafe = jnp.where(inr, ploc, 0)
            cnt, last = plsc.scan_count(pidv, inr)
            c0 = plsc.load_gather(counter, [psafe], mask=inr)
            rank = c0 + cnt - cbase
            keep = inr & (rank < MAXPTS)
            rsafe = jnp.where(keep, rank, 0)
            xv = x_st[pl.ds(v * 16, 16)]
            yv = y_st[pl.ds(v * 16, 16)]
            zv = z_st[pl.ds(v * 16, 16)]
            plsc.store_scatter(valx, [rsafe, psafe], xv, mask=keep)
            plsc.store_scatter(valy, [rsafe, psafe], yv, mask=keep)
            plsc.store_scatter(valz, [rsafe, psafe], zv, mask=keep)
            plsc.store_scatter(counter, [psafe], c0 + cnt + 1 - cbase,
                               mask=last & inr)
            return 0
        lax.fori_loop(0, VECS, _vec, 0)

    _start(0, pidA, xA, yA, zA, semA)

    def _pair(i, _):
        _drain(pidA, xA, yA, zA, semA)
        _start(2 * i + 1, pidB, xB, yB, zB, semB)
        _process(pidA, xA, yA, zA)
        _drain(pidB, xB, yB, zB, semB)

        @pl.when(i < NCH // 2 - 1)
        def _():
            _start(2 * i + 2, pidA, xA, yA, zA, semA)
        _process(pidB, xB, yB, zB)
        return 0
    lax.fori_loop(0, NCH // 2, _pair, 0)

    def _cap(k, _):
        v = counter[pl.ds(k * 16, 16)]
        counter[pl.ds(k * 16, 16)] = jnp.minimum(v, MAXPTS)
        return 0
    lax.fori_loop(0, RNG // 16, _cap, 0)
    pltpu.sync_copy(counter, counts_hbm.at[pl.ds(cnt_base, RNG)])

    pltpu.sync_copy(valx, buf_hbm.at[b, 0, :, pl.ds(base_p, RNG)])
    pltpu.sync_copy(valy, buf_hbm.at[b, 1, :, pl.ds(base_p, RNG)])
    pltpu.sync_copy(valz, buf_hbm.at[b, 2, :, pl.ds(base_p, RNG)])


def _sc_scatter(pid1, xf, yf, zf):
    mesh = plsc.VectorSubcoreMesh(core_axis_name="c", subcore_axis_name="s")
    f = functools.partial(
        pl.kernel, mesh=mesh,
        out_type=(
            jax.ShapeDtypeStruct((BB, 3, MAXPTS, PPAD), jnp.float32),
            jax.ShapeDtypeStruct((BB * PPAD,), jnp.int32),
        ),
        scratch_types=[
            pltpu.VMEM((CH,), jnp.int32),
            pltpu.VMEM((CH,), jnp.int32),
            pltpu.VMEM((CH,), jnp.float32),
            pltpu.VMEM((CH,), jnp.float32),
            pltpu.VMEM((CH,), jnp.float32),
            pltpu.VMEM((CH,), jnp.float32),
            pltpu.VMEM((CH,), jnp.float32),
            pltpu.VMEM((CH,), jnp.float32),
            pltpu.VMEM((RNG,), jnp.int32),
            pltpu.VMEM((MAXPTS, RNG), jnp.float32),
            pltpu.VMEM((MAXPTS, RNG), jnp.float32),
            pltpu.VMEM((MAXPTS, RNG), jnp.float32),
            pltpu.SemaphoreType.DMA,
            pltpu.SemaphoreType.DMA,
        ],
        compiler_params=pltpu.CompilerParams(needs_layout_passes=False),
    )(_sc_body)
    return f(pid1, xf, yf, zf)


def _enc_body(buf_ref, cnt_ref, xc_ref, yc_ref, w_ref, b_ref, gam_ref,
              bet_ref, mu_ref, var_ref, out_ref):
    x = buf_ref[0, 0]
    y = buf_ref[0, 1]
    z = buf_ref[0, 2]
    cnt2 = cnt_ref[0, 0]
    slot = jax.lax.broadcasted_iota(jnp.int32, (MAXPTS, PPAD), 0)
    mask = slot < cnt2
    cntf = jnp.maximum(cnt2, 1).astype(jnp.float32)
    mx = jnp.sum(jnp.where(mask, x, 0.0), 0, keepdims=True) / cntf
    my = jnp.sum(jnp.where(mask, y, 0.0), 0, keepdims=True) / cntf
    mz = jnp.sum(jnp.where(mask, z, 0.0), 0, keepdims=True) / cntf
    xc = xc_ref[0]
    yc = yc_ref[0]

    sc_ = gam_ref[0] / jnp.sqrt(var_ref[0] + 1e-5)
    w0 = w_ref[0, 0] * sc_
    w1 = w_ref[0, 1] * sc_
    w2 = w_ref[0, 2] * sc_
    w3 = w_ref[0, 3] * sc_
    w4 = w_ref[0, 4] * sc_
    w5 = w_ref[0, 5] * sc_
    w6 = w_ref[0, 6] * sc_
    w7 = w_ref[0, 7] * sc_
    c0 = b_ref[0] * sc_ + bet_ref[0] - mu_ref[0] * sc_
    a3 = jnp.concatenate([w0 + w3 + w6, w1 + w4 + w7, w2 + w5], axis=1)
    w6m = jnp.concatenate([-w3, -w4, -w5, -w6, -w7, c0], axis=1)

    ones = jnp.ones((1, PPAD), jnp.float32)
    m6 = jnp.concatenate([mx, my, mz, xc, yc, ones], axis=0)
    cc = jnp.dot(w6m, m6, precision=lax.Precision.HIGHEST)

    neg = jnp.float32(-1e9)
    m = jnp.full((COUT, PPAD), neg, jnp.float32)
    for i in range(MAXPTS):
        xyz_i = jnp.concatenate(
            [x[i:i + 1], y[i:i + 1], z[i:i + 1]], axis=0)
        h_i = jnp.dot(a3, xyz_i, precision=lax.Precision.HIGHEST)
        m = jnp.maximum(m, jnp.where(cnt2 > i, h_i, neg))
    res = jnp.where(cnt2 > 0, jnp.maximum(m + cc, 0.0), 0.0)
    out_ref[0] = res[:, :NP]


def _encode(buf, counts4, xc3, yc3, w3d, bb, gamma, beta, mu, var):
    vspec = pl.BlockSpec((1, 8, COUT, 1), lambda bi: (0, 0, 0, 0))
    pspec = pl.BlockSpec((1, COUT, 1), lambda bi: (0, 0, 0))
    return pl.pallas_call(
        _enc_body,
        grid=(BB,),
        in_specs=[
            pl.BlockSpec((1, 3, MAXPTS, PPAD), lambda bi: (bi, 0, 0, 0)),
            pl.BlockSpec((1, 1, 1, PPAD), lambda bi: (bi, 0, 0, 0)),
            pl.BlockSpec((1, 1, PPAD), lambda bi: (0, 0, 0)),
            pl.BlockSpec((1, 1, PPAD), lambda bi: (0, 0, 0)),
            vspec, pspec, pspec, pspec, pspec, pspec,
        ],
        out_specs=pl.BlockSpec((1, COUT, NP), lambda bi: (bi, 0, 0)),
        out_shape=jax.ShapeDtypeStruct((BB, COUT, NP), jnp.float32),
    )(buf, counts4, xc3, yc3, w3d, bb, gamma, beta, mu, var)


def kernel(batched_pts, W, b, gamma, beta, bn_mean, bn_var):
    ptsT = jnp.pad(batched_pts.transpose(0, 2, 1),
                   ((0, 0), (0, 0), (0, NPAD - NPTS)),
                   constant_values=-1e9)
    pts4d = ptsT.reshape(BB, 4, NPAD // 128, 128)

    pid1 = _compute_pids(pts4d).reshape(BB * NPAD)
    xf = ptsT[:, 0, :].reshape(BB * NPAD)
    yf = ptsT[:, 1, :].reshape(BB * NPAD)
    zf = ptsT[:, 2, :].reshape(BB * NPAD)

    buf, counts = _sc_scatter(pid1, xf, yf, zf)

    p_idx = jnp.arange(PPAD, dtype=jnp.int32)
    xc = ((p_idx % NX).astype(jnp.float32) + 0.5) * VX + XMIN
    yc = ((p_idx // NX).astype(jnp.float32) + 0.5) * VY + YMIN

    out = _encode(buf, counts.reshape(BB, 1, 1, PPAD),
                  xc.reshape(1, 1, PPAD), yc.reshape(1, 1, PPAD),
                  W.reshape(1, 8, COUT, 1), b.reshape(1, COUT, 1),
                  gamma.reshape(1, COUT, 1), beta.reshape(1, COUT, 1),
                  bn_mean.reshape(1, COUT, 1), bn_var.reshape(1, COUT, 1))

    return out.reshape(BB, COUT, NY, NX)
